# baseline (speedup 1.0000x reference)
import jax
import jax.numpy as jnp
from jax import lax
from jax.experimental import pallas as pl
from jax.experimental.pallas import tpu as pltpu
from jax.experimental.pallas import tpu_sc as plsc

N = 50000
E = 800000
NP = 51200
H = 64
QH = 16
G = 64
C = 5
NPR = NP // 8
RB = 640
NBL = NPR // RB
CH = 1000
EPW = E // 32
EPS = E // 16
SCH = 2000
NCHS = EPS // SCH
RPS = NP // 16
ZR = 320
PR = 320
PCH = 800

_mesh = plsc.VectorSubcoreMesh(core_axis_name="c", subcore_axis_name="s")
_sc_params = pltpu.CompilerParams(use_tc_tiling_on_sc=False)


def _sc_hist_body(dst_hbm, hist_hbm, idst0, idst1, ones_v, z1, acc1,
                  ssem0, ssem1):
    cid = lax.axis_index("c")
    sid = lax.axis_index("s")
    wid = sid * 2 + cid
    idst = (idst0, idst1)
    ssem = (ssem0, ssem1)

    def idst_load(j, k):
        base = wid * EPW + k * CH
        pltpu.sync_copy(dst_hbm.at[pl.ds(base, CH)], idst[j])

    def scat_start(j):
        pltpu.async_copy(ones_v.at[pl.ds(0, CH)], acc1.at[idst[j]], ssem[j],
                         add=True)

    def scat_wait(j):
        pltpu.make_async_copy(ones_v.at[pl.ds(0, CH)], acc1.at[idst[j]],
                              ssem[j]).wait()

    @pl.loop(0, 64)
    def _(j):
        ones_v[pl.ds(j * 16, 16)] = jnp.ones((16,), jnp.float32)

    @pl.loop(0, ZR // 16)
    def _(j):
        z1[pl.ds(j * 16, 16)] = jnp.zeros((16,), jnp.float32)

    idst_load(0, 0)

    @pl.loop(0, RPS // ZR)
    def _(j):
        pltpu.sync_copy(z1, acc1.at[pl.ds(sid * RPS + j * ZR, ZR)])

    plsc.subcore_barrier()

    scat_start(0)

    @pl.loop(0, (EPW // CH - 1) // 2)
    def _(i):
        idst_load(1, 2 * i + 1)
        scat_wait(0)
        scat_start(1)
        idst_load(0, 2 * i + 2)
        scat_wait(1)
        scat_start(0)

    scat_wait(0)

    plsc.subcore_barrier()
    pltpu.sync_copy(acc1.at[pl.ds(sid * RPS, RPS)],
                    hist_hbm.at[pl.ds(cid * NP + sid * RPS, RPS)])


@jax.jit
def _sc_hist(dst):
    k = pl.kernel(
        _sc_hist_body,
        out_type=jax.ShapeDtypeStruct((2 * NP,), jnp.float32),
        mesh=_mesh,
        scratch_types=[
            pltpu.VMEM((CH,), jnp.int32),
            pltpu.VMEM((CH,), jnp.int32),
            pltpu.VMEM((1024,), jnp.float32),
            pltpu.VMEM((ZR,), jnp.float32),
            pltpu.VMEM_SHARED((NP,), jnp.float32),
            pltpu.SemaphoreType.DMA,
            pltpu.SemaphoreType.DMA,
        ],
        compiler_params=_sc_params,
    )
    return k(dst)


def _sc_scatter_body(src4_hbm, dst_hbm, u_hbm, s_hbm,
                     isrc0, isrc1, idst0, idst1, rows0, rows1, zbuf, acc,
                     gsem0, gsem1, isem0, isem1):
    cid = lax.axis_index("c")
    sid = lax.axis_index("s")
    isrc = (isrc0, isrc1)
    idst = (idst0, idst1)
    rows = (rows0, rows1)
    gsem = (gsem0, gsem1)
    isem = (isem0, isem1)

    @pl.loop(0, ZR)
    def _(r):
        zbuf[r, pl.ds(0, 16)] = jnp.zeros((16,), jnp.float32)

    @pl.loop(0, 2)
    def _(p):
        q = 2 * cid + p

        def isrc_start(j, k):
            kk = jnp.where(k < NCHS, k, 0)
            base = sid * EPS + kk * SCH
            pltpu.async_copy(src4_hbm.at[pl.ds(q * E + base, SCH)], isrc[j],
                             isem[j])

        def idst_start(j, k):
            kk = jnp.where(k < NCHS, k, 0)
            base = sid * EPS + kk * SCH
            pltpu.async_copy(dst_hbm.at[pl.ds(base, SCH)], idst[j], isem[j])

        def idx_wait(j):
            pltpu.make_async_copy(src4_hbm.at[pl.ds(0, SCH)], isrc[j],
                                  isem[j]).wait()
            pltpu.make_async_copy(dst_hbm.at[pl.ds(0, SCH)], idst[j],
                                  isem[j]).wait()

        def gather_start(j):
            hs = SCH // 2
            pltpu.async_copy(u_hbm.at[isrc[j].at[pl.ds(0, hs)]],
                             rows[j].at[pl.ds(0, hs)], gsem[j])
            pltpu.async_copy(u_hbm.at[isrc[j].at[pl.ds(hs, hs)]],
                             rows[j].at[pl.ds(hs, hs)], gsem[j])

        def gather_wait(j):
            hs = SCH // 2
            pltpu.make_async_copy(u_hbm.at[isrc[j].at[pl.ds(0, hs)]],
                                  rows[j].at[pl.ds(0, hs)], gsem[j]).wait()
            pltpu.make_async_copy(u_hbm.at[isrc[j].at[pl.ds(hs, hs)]],
                                  rows[j].at[pl.ds(hs, hs)], gsem[j]).wait()

        def scatter(j):
            pltpu.sync_copy(rows[j], acc.at[idst[j]], add=True)

        isrc_start(0, 0)
        idst_start(0, 0)
        isrc_start(1, 1)
        idst_start(1, 1)

        @pl.loop(0, RPS // ZR)
        def _(j):
            pltpu.sync_copy(zbuf, acc.at[pl.ds(sid * RPS + j * ZR, ZR)])

        idx_wait(0)
        gather_start(0)
        plsc.subcore_barrier()

        @pl.loop(0, (NCHS - 1) // 2)
        def _(i):
            idx_wait(1)
            gather_start(1)
            gather_wait(0)
            isrc_start(0, 2 * i + 2)
            scatter(0)
            idst_start(0, 2 * i + 2)
            idx_wait(0)
            gather_start(0)
            gather_wait(1)
            isrc_start(1, 2 * i + 3)
            scatter(1)
            idst_start(1, 2 * i + 3)

        gather_wait(0)
        scatter(0)
        idx_wait(1)

        plsc.subcore_barrier()
        pltpu.sync_copy(acc.at[pl.ds(sid * RPS, RPS)],
                        s_hbm.at[pl.ds(q * NP + sid * RPS, RPS)])


@jax.jit
def _sc_scatter(src4, dst, u):
    k = pl.kernel(
        _sc_scatter_body,
        out_type=jax.ShapeDtypeStruct((4 * NP, QH), jnp.float32),
        mesh=_mesh,
        scratch_types=[
            pltpu.VMEM((SCH,), jnp.int32),
            pltpu.VMEM((SCH,), jnp.int32),
            pltpu.VMEM((SCH,), jnp.int32),
            pltpu.VMEM((SCH,), jnp.int32),
            pltpu.VMEM((SCH, QH), jnp.float32),
            pltpu.VMEM((SCH, QH), jnp.float32),
            pltpu.VMEM((ZR, QH), jnp.float32),
            pltpu.VMEM_SHARED((NP, QH), jnp.float32),
            pltpu.SemaphoreType.DMA,
            pltpu.SemaphoreType.DMA,
            pltpu.SemaphoreType.DMA,
            pltpu.SemaphoreType.DMA,
        ],
        compiler_params=_sc_params,
    )
    return k(src4, dst, u)


def _a1_body(x_ref, w_ref, h0_ref, h1_ref, r_ref, o_ref, dinv_ref):
    dinv8 = lax.rsqrt(h0_ref[...] + h1_ref[...] + 1.0)
    dinv16 = jnp.dot(dinv8, r_ref[...], preferred_element_type=jnp.float32)
    dinv_ref[...] = dinv16
    o_ref[...] = jnp.dot(x_ref[...], w_ref[0],
                         preferred_element_type=jnp.float32) * dinv16


@jax.jit
def _a1(x_packed, Wbd1, h0_8, h1_8, R):
    return pl.pallas_call(
        _a1_body,
        grid=(NBL, 4),
        in_specs=[
            pl.BlockSpec((RB, 64), lambda i, q: (i, 0)),
            pl.BlockSpec((1, 64, 128), lambda i, q: (q, 0, 0)),
            pl.BlockSpec((RB, 8), lambda i, q: (i, 0)),
            pl.BlockSpec((RB, 8), lambda i, q: (i, 0)),
            pl.BlockSpec((8, 128), lambda i, q: (0, 0)),
        ],
        out_specs=[
            pl.BlockSpec((RB, 128), lambda i, q: (q * NBL + i, 0)),
            pl.BlockSpec((RB, 128), lambda i, q: (i, 0)),
        ],
        out_shape=[
            jax.ShapeDtypeStruct((4 * NPR, 128), jnp.float32),
            jax.ShapeDtypeStruct((NPR, 128), jnp.float32),
        ],
    )(x_packed, Wbd1, h0_8, h1_8, R)


def _h_quarters(s_refs, u_refs, dinv, b4):
    return [jnp.maximum((s_refs[q][...] + u_refs[q][...]) * dinv
                        + b4[q:q + 1, :], 0.0) for q in range(4)]


def _fused_body(s0, s1, s2, s3, u0, u1, u2, u3, dinv_ref, b4_ref, w_ref,
                o_ref, h_s):
    qo = pl.program_id(1)
    dinv = dinv_ref[...]

    @pl.when(qo == 0)
    def _():
        hs = _h_quarters((s0, s1, s2, s3), (u0, u1, u2, u3), dinv, b4_ref[...])
        for q in range(4):
            h_s[q] = hs[q]

    wb = w_ref[0]
    acc = jnp.dot(h_s[0], wb[0:128], preferred_element_type=jnp.float32)
    for q in range(1, 4):
        acc += jnp.dot(h_s[q], wb[128 * q:128 * (q + 1)],
                       preferred_element_type=jnp.float32)
    o_ref[...] = acc * dinv


@jax.jit
def _fused(s, u, dinv16, b4, Wbd):
    qspecs = [pl.BlockSpec((RB, 128), (lambda q: (lambda i, qo: (q * NBL + i, 0)))(q))
              for q in range(4)]
    return pl.pallas_call(
        _fused_body,
        grid=(NBL, 4),
        in_specs=qspecs + qspecs + [
            pl.BlockSpec((RB, 128), lambda i, qo: (i, 0)),
            pl.BlockSpec((4, 128), lambda i, qo: (0, 0)),
            pl.BlockSpec((1, 512, 128), lambda i, qo: (qo, 0, 0)),
        ],
        out_specs=pl.BlockSpec((RB, 128), lambda i, qo: (qo * NBL + i, 0)),
        out_shape=jax.ShapeDtypeStruct((4 * NPR, 128), jnp.float32),
        scratch_shapes=[pltpu.VMEM((4, RB, 128), jnp.float32)],
    )(s, s, s, s, u, u, u, u, dinv16, b4, Wbd)


def _h3pool_body(s0, s1, s2, s3, u0, u1, u2, u3, dinv_ref, b4_ref, bt_ref,
                 wl_ref, bl_ref, o_ref, acc):
    i = pl.program_id(0)

    @pl.when(i == 0)
    def _():
        acc[...] = jnp.zeros((G, 128), jnp.float32)

    hq = _h_quarters((s0, s1, s2, s3), (u0, u1, u2, u3), dinv_ref[...],
                     b4_ref[...])
    bt = bt_ref[...]
    iota_g = lax.broadcasted_iota(jnp.int32, (RB, G), 1)
    ones_blk = jnp.ones((RB, G), jnp.float32)
    a = acc[...]
    for sub in range(8):
        oh = (bt[:, sub:sub + 1] == iota_g).astype(jnp.float32)
        hcat = jnp.concatenate(
            [hq[q][:, QH * sub:QH * (sub + 1)] for q in range(4)]
            + [ones_blk], axis=1)
        a += lax.dot_general(oh, hcat, (((0,), (0,)), ((), ())),
                             preferred_element_type=jnp.float32)
    acc[...] = a

    @pl.when(i == NBL - 1)
    def _():
        fp = a[:, :H] / jnp.maximum(a[:, H:], 1.0)
        o_ref[...] = jnp.dot(fp, wl_ref[...],
                             preferred_element_type=jnp.float32) + bl_ref[...]


@jax.jit
def _h3pool(s, u, dinv16, b4, batch_pk, Wl, bl2):
    qspecs = [pl.BlockSpec((RB, 128), (lambda q: (lambda i: (q * NBL + i, 0)))(q))
              for q in range(4)]
    return pl.pallas_call(
        _h3pool_body,
        grid=(NBL,),
        in_specs=qspecs + qspecs + [
            pl.BlockSpec((RB, 128), lambda i: (i, 0)),
            pl.BlockSpec((4, 128), lambda i: (0, 0)),
            pl.BlockSpec((RB, 8), lambda i: (i, 0)),
            pl.BlockSpec((H, C), lambda i: (0, 0)),
            pl.BlockSpec((1, C), lambda i: (0, 0)),
        ],
        out_specs=pl.BlockSpec((G, C), lambda i: (0, 0)),
        out_shape=jax.ShapeDtypeStruct((G, C), jnp.float32),
        scratch_shapes=[pltpu.VMEM((G, 128), jnp.float32)],
    )(s, s, s, s, u, u, u, u, dinv16, b4, batch_pk, Wl, bl2)


def kernel(x, edge_index, batch, W1, b1, W2, b2, W3, b3, Wl, bl):
    f32 = jnp.float32
    src = edge_index[0]
    dst = edge_index[1]
    src4 = jnp.concatenate([src + q * NP for q in range(4)])
    batch_pk = jnp.pad(batch, (0, NP - N), constant_values=G).reshape(NPR, 8)
    x_packed = jnp.pad(x, ((0, NP - N), (0, 5))).reshape(NPR, 64)

    eye8 = jnp.eye(8, dtype=f32)
    R = jnp.kron(eye8, jnp.ones((1, QH), f32))

    def bd1(q):
        wq = jnp.pad(W1[:, QH * q:QH * (q + 1)], ((0, 5), (0, 0)))
        return jnp.kron(eye8, wq)

    Wbd1 = jnp.stack([bd1(q) for q in range(4)])

    def bd(W):
        return jnp.stack([
            jnp.concatenate(
                [jnp.kron(eye8, W[QH * qi:QH * (qi + 1), QH * qo:QH * (qo + 1)])
                 for qi in range(4)], axis=0)
            for qo in range(4)])

    def pack_bias(b):
        return jnp.tile(b.reshape(4, QH), (1, 8))

    hist = _sc_hist(dst)
    u, dinv16 = _a1(x_packed, Wbd1, hist[:NP].reshape(NPR, 8),
                    hist[NP:].reshape(NPR, 8), R)
    s = _sc_scatter(src4, dst, u.reshape(4 * NP, QH))
    u = _fused(s.reshape(4 * NPR, 128), u, dinv16, pack_bias(b1), bd(W2))
    s = _sc_scatter(src4, dst, u.reshape(4 * NP, QH))
    u = _fused(s.reshape(4 * NPR, 128), u, dinv16, pack_bias(b2), bd(W3))
    s = _sc_scatter(src4, dst, u.reshape(4 * NP, QH))
    return _h3pool(s.reshape(4 * NPR, 128), u, dinv16, pack_bias(b3),
                   batch_pk, Wl, bl.reshape(1, C))


# speedup vs baseline: 1.0005x; 1.0005x over previous
import jax
import jax.numpy as jnp
from jax import lax
from jax.experimental import pallas as pl
from jax.experimental.pallas import tpu as pltpu
from jax.experimental.pallas import tpu_sc as plsc

N = 50000
E = 800000
NP = 51200
H = 64
QH = 16
G = 64
C = 5
NPR = NP // 8
RB = 640
NBL = NPR // RB
CH = 1000
EPW = E // 32
EPS = E // 16
SCH = 2000
NCHS = EPS // SCH
RPS = NP // 16
ZR = 320
PR = 320
PCH = 800

_mesh = plsc.VectorSubcoreMesh(core_axis_name="c", subcore_axis_name="s")
_sc_params = pltpu.CompilerParams(use_tc_tiling_on_sc=False)


def _sc_hist_body(dst_hbm, hist_hbm, idst0, idst1, ones_v, z1, acc1,
                  ssem0, ssem1):
    cid = lax.axis_index("c")
    sid = lax.axis_index("s")
    wid = sid * 2 + cid
    idst = (idst0, idst1)
    ssem = (ssem0, ssem1)

    def idst_load(j, k):
        base = wid * EPW + k * CH
        pltpu.sync_copy(dst_hbm.at[pl.ds(base, CH)], idst[j])

    def scat_start(j):
        pltpu.async_copy(ones_v.at[pl.ds(0, CH)], acc1.at[idst[j]], ssem[j],
                         add=True)

    def scat_wait(j):
        pltpu.make_async_copy(ones_v.at[pl.ds(0, CH)], acc1.at[idst[j]],
                              ssem[j]).wait()

    @pl.loop(0, 64)
    def _(j):
        ones_v[pl.ds(j * 16, 16)] = jnp.ones((16,), jnp.float32)

    @pl.loop(0, ZR // 16)
    def _(j):
        z1[pl.ds(j * 16, 16)] = jnp.zeros((16,), jnp.float32)

    idst_load(0, 0)

    @pl.loop(0, RPS // ZR)
    def _(j):
        pltpu.sync_copy(z1, acc1.at[pl.ds(sid * RPS + j * ZR, ZR)])

    plsc.subcore_barrier()

    scat_start(0)

    @pl.loop(0, (EPW // CH - 1) // 2)
    def _(i):
        idst_load(1, 2 * i + 1)
        scat_wait(0)
        scat_start(1)
        idst_load(0, 2 * i + 2)
        scat_wait(1)
        scat_start(0)

    scat_wait(0)

    plsc.subcore_barrier()
    pltpu.sync_copy(acc1.at[pl.ds(sid * RPS, RPS)],
                    hist_hbm.at[pl.ds(cid * NP + sid * RPS, RPS)])


@jax.jit
def _sc_hist(dst):
    k = pl.kernel(
        _sc_hist_body,
        out_type=jax.ShapeDtypeStruct((2 * NP,), jnp.float32),
        mesh=_mesh,
        scratch_types=[
            pltpu.VMEM((CH,), jnp.int32),
            pltpu.VMEM((CH,), jnp.int32),
            pltpu.VMEM((1024,), jnp.float32),
            pltpu.VMEM((ZR,), jnp.float32),
            pltpu.VMEM_SHARED((NP,), jnp.float32),
            pltpu.SemaphoreType.DMA,
            pltpu.SemaphoreType.DMA,
        ],
        compiler_params=_sc_params,
    )
    return k(dst)


def _sc_scatter_body(src4_hbm, dst_hbm, u_hbm, s_hbm,
                     isrc0, isrc1, idst0, idst1, rows0, rows1, zbuf, acc,
                     gsem0, gsem1, isem0, isem1):
    cid = lax.axis_index("c")
    sid = lax.axis_index("s")
    isrc = (isrc0, isrc1)
    idst = (idst0, idst1)
    rows = (rows0, rows1)
    gsem = (gsem0, gsem1)
    isem = (isem0, isem1)

    @pl.loop(0, ZR)
    def _(r):
        zbuf[r, pl.ds(0, 16)] = jnp.zeros((16,), jnp.float32)

    @pl.loop(0, 2)
    def _(p):
        q = 2 * cid + p

        def isrc_start(j, k):
            kk = jnp.where(k < NCHS, k, 0)
            base = sid * EPS + kk * SCH
            pltpu.async_copy(src4_hbm.at[pl.ds(q * E + base, SCH)], isrc[j],
                             isem[j])

        def idst_start(j, k):
            kk = jnp.where(k < NCHS, k, 0)
            base = sid * EPS + kk * SCH
            pltpu.async_copy(dst_hbm.at[pl.ds(base, SCH)], idst[j], isem[j])

        def idx_wait(j):
            pltpu.make_async_copy(src4_hbm.at[pl.ds(0, SCH)], isrc[j],
                                  isem[j]).wait()
            pltpu.make_async_copy(dst_hbm.at[pl.ds(0, SCH)], idst[j],
                                  isem[j]).wait()

        def gather_start(j):
            pltpu.async_copy(u_hbm.at[isrc[j]], rows[j], gsem[j])

        def gather_wait(j):
            pltpu.make_async_copy(u_hbm.at[isrc[j]], rows[j], gsem[j]).wait()

        def scatter(j):
            pltpu.sync_copy(rows[j], acc.at[idst[j]], add=True)

        isrc_start(0, 0)
        idst_start(0, 0)
        isrc_start(1, 1)
        idst_start(1, 1)

        @pl.loop(0, RPS // ZR)
        def _(j):
            pltpu.sync_copy(zbuf, acc.at[pl.ds(sid * RPS + j * ZR, ZR)])

        idx_wait(0)
        gather_start(0)
        plsc.subcore_barrier()

        @pl.loop(0, (NCHS - 1) // 2)
        def _(i):
            idx_wait(1)
            gather_start(1)
            gather_wait(0)
            isrc_start(0, 2 * i + 2)
            scatter(0)
            idst_start(0, 2 * i + 2)
            idx_wait(0)
            gather_start(0)
            gather_wait(1)
            isrc_start(1, 2 * i + 3)
            scatter(1)
            idst_start(1, 2 * i + 3)

        gather_wait(0)
        scatter(0)
        idx_wait(1)

        plsc.subcore_barrier()
        pltpu.sync_copy(acc.at[pl.ds(sid * RPS, RPS)],
                        s_hbm.at[pl.ds(q * NP + sid * RPS, RPS)])


@jax.jit
def _sc_scatter(src4, dst, u):
    k = pl.kernel(
        _sc_scatter_body,
        out_type=jax.ShapeDtypeStruct((4 * NP, QH), jnp.float32),
        mesh=_mesh,
        scratch_types=[
            pltpu.VMEM((SCH,), jnp.int32),
            pltpu.VMEM((SCH,), jnp.int32),
            pltpu.VMEM((SCH,), jnp.int32),
            pltpu.VMEM((SCH,), jnp.int32),
            pltpu.VMEM((SCH, QH), jnp.float32),
            pltpu.VMEM((SCH, QH), jnp.float32),
            pltpu.VMEM((ZR, QH), jnp.float32),
            pltpu.VMEM_SHARED((NP, QH), jnp.float32),
            pltpu.SemaphoreType.DMA,
            pltpu.SemaphoreType.DMA,
            pltpu.SemaphoreType.DMA,
            pltpu.SemaphoreType.DMA,
        ],
        compiler_params=_sc_params,
    )
    return k(src4, dst, u)


def _a1_body(x_ref, w_ref, h0_ref, h1_ref, r_ref, o_ref, dinv_ref):
    dinv8 = lax.rsqrt(h0_ref[...] + h1_ref[...] + 1.0)
    dinv16 = jnp.dot(dinv8, r_ref[...], preferred_element_type=jnp.float32)
    dinv_ref[...] = dinv16
    o_ref[...] = jnp.dot(x_ref[...], w_ref[0],
                         preferred_element_type=jnp.float32) * dinv16


@jax.jit
def _a1(x_packed, Wbd1, h0_8, h1_8, R):
    return pl.pallas_call(
        _a1_body,
        grid=(NBL, 4),
        in_specs=[
            pl.BlockSpec((RB, 64), lambda i, q: (i, 0)),
            pl.BlockSpec((1, 64, 128), lambda i, q: (q, 0, 0)),
            pl.BlockSpec((RB, 8), lambda i, q: (i, 0)),
            pl.BlockSpec((RB, 8), lambda i, q: (i, 0)),
            pl.BlockSpec((8, 128), lambda i, q: (0, 0)),
        ],
        out_specs=[
            pl.BlockSpec((RB, 128), lambda i, q: (q * NBL + i, 0)),
            pl.BlockSpec((RB, 128), lambda i, q: (i, 0)),
        ],
        out_shape=[
            jax.ShapeDtypeStruct((4 * NPR, 128), jnp.float32),
            jax.ShapeDtypeStruct((NPR, 128), jnp.float32),
        ],
    )(x_packed, Wbd1, h0_8, h1_8, R)


def _h_quarters(s_refs, u_refs, dinv, b4):
    return [jnp.maximum((s_refs[q][...] + u_refs[q][...]) * dinv
                        + b4[q:q + 1, :], 0.0) for q in range(4)]


def _fused_body(s0, s1, s2, s3, u0, u1, u2, u3, dinv_ref, b4_ref, w_ref,
                o_ref, h_s):
    qo = pl.program_id(1)
    dinv = dinv_ref[...]

    @pl.when(qo == 0)
    def _():
        hs = _h_quarters((s0, s1, s2, s3), (u0, u1, u2, u3), dinv, b4_ref[...])
        for q in range(4):
            h_s[q] = hs[q]

    wb = w_ref[0]
    acc = jnp.dot(h_s[0], wb[0:128], preferred_element_type=jnp.float32)
    for q in range(1, 4):
        acc += jnp.dot(h_s[q], wb[128 * q:128 * (q + 1)],
                       preferred_element_type=jnp.float32)
    o_ref[...] = acc * dinv


@jax.jit
def _fused(s, u, dinv16, b4, Wbd):
    qspecs = [pl.BlockSpec((RB, 128), (lambda q: (lambda i, qo: (q * NBL + i, 0)))(q))
              for q in range(4)]
    return pl.pallas_call(
        _fused_body,
        grid=(NBL, 4),
        in_specs=qspecs + qspecs + [
            pl.BlockSpec((RB, 128), lambda i, qo: (i, 0)),
            pl.BlockSpec((4, 128), lambda i, qo: (0, 0)),
            pl.BlockSpec((1, 512, 128), lambda i, qo: (qo, 0, 0)),
        ],
        out_specs=pl.BlockSpec((RB, 128), lambda i, qo: (qo * NBL + i, 0)),
        out_shape=jax.ShapeDtypeStruct((4 * NPR, 128), jnp.float32),
        scratch_shapes=[pltpu.VMEM((4, RB, 128), jnp.float32)],
    )(s, s, s, s, u, u, u, u, dinv16, b4, Wbd)


def _h3pool_body(s0, s1, s2, s3, u0, u1, u2, u3, dinv_ref, b4_ref, bt_ref,
                 wl_ref, bl_ref, o_ref, acc):
    i = pl.program_id(0)

    @pl.when(i == 0)
    def _():
        acc[...] = jnp.zeros((G, 128), jnp.float32)

    hq = _h_quarters((s0, s1, s2, s3), (u0, u1, u2, u3), dinv_ref[...],
                     b4_ref[...])
    bt = bt_ref[...]
    iota_g = lax.broadcasted_iota(jnp.int32, (RB, G), 1)
    ones_blk = jnp.ones((RB, G), jnp.float32)
    a = acc[...]
    for sub in range(8):
        oh = (bt[:, sub:sub + 1] == iota_g).astype(jnp.float32)
        hcat = jnp.concatenate(
            [hq[q][:, QH * sub:QH * (sub + 1)] for q in range(4)]
            + [ones_blk], axis=1)
        a += lax.dot_general(oh, hcat, (((0,), (0,)), ((), ())),
                             preferred_element_type=jnp.float32)
    acc[...] = a

    @pl.when(i == NBL - 1)
    def _():
        fp = a[:, :H] / jnp.maximum(a[:, H:], 1.0)
        o_ref[...] = jnp.dot(fp, wl_ref[...],
                             preferred_element_type=jnp.float32) + bl_ref[...]


@jax.jit
def _h3pool(s, u, dinv16, b4, batch_pk, Wl, bl2):
    qspecs = [pl.BlockSpec((RB, 128), (lambda q: (lambda i: (q * NBL + i, 0)))(q))
              for q in range(4)]
    return pl.pallas_call(
        _h3pool_body,
        grid=(NBL,),
        in_specs=qspecs + qspecs + [
            pl.BlockSpec((RB, 128), lambda i: (i, 0)),
            pl.BlockSpec((4, 128), lambda i: (0, 0)),
            pl.BlockSpec((RB, 8), lambda i: (i, 0)),
            pl.BlockSpec((H, C), lambda i: (0, 0)),
            pl.BlockSpec((1, C), lambda i: (0, 0)),
        ],
        out_specs=pl.BlockSpec((G, C), lambda i: (0, 0)),
        out_shape=jax.ShapeDtypeStruct((G, C), jnp.float32),
        scratch_shapes=[pltpu.VMEM((G, 128), jnp.float32)],
    )(s, s, s, s, u, u, u, u, dinv16, b4, batch_pk, Wl, bl2)


def kernel(x, edge_index, batch, W1, b1, W2, b2, W3, b3, Wl, bl):
    f32 = jnp.float32
    src = edge_index[0]
    dst = edge_index[1]
    src4 = jnp.concatenate([src + q * NP for q in range(4)])
    batch_pk = jnp.pad(batch, (0, NP - N), constant_values=G).reshape(NPR, 8)
    x_packed = jnp.pad(x, ((0, NP - N), (0, 5))).reshape(NPR, 64)

    eye8 = jnp.eye(8, dtype=f32)
    R = jnp.kron(eye8, jnp.ones((1, QH), f32))

    def bd1(q):
        wq = jnp.pad(W1[:, QH * q:QH * (q + 1)], ((0, 5), (0, 0)))
        return jnp.kron(eye8, wq)

    Wbd1 = jnp.stack([bd1(q) for q in range(4)])

    def bd(W):
        return jnp.stack([
            jnp.concatenate(
                [jnp.kron(eye8, W[QH * qi:QH * (qi + 1), QH * qo:QH * (qo + 1)])
                 for qi in range(4)], axis=0)
            for qo in range(4)])

    def pack_bias(b):
        return jnp.tile(b.reshape(4, QH), (1, 8))

    hist = _sc_hist(dst)
    u, dinv16 = _a1(x_packed, Wbd1, hist[:NP].reshape(NPR, 8),
                    hist[NP:].reshape(NPR, 8), R)
    s = _sc_scatter(src4, dst, u.reshape(4 * NP, QH))
    u = _fused(s.reshape(4 * NPR, 128), u, dinv16, pack_bias(b1), bd(W2))
    s = _sc_scatter(src4, dst, u.reshape(4 * NP, QH))
    u = _fused(s.reshape(4 * NPR, 128), u, dinv16, pack_bias(b2), bd(W3))
    s = _sc_scatter(src4, dst, u.reshape(4 * NP, QH))
    return _h3pool(s.reshape(4 * NPR, 128), u, dinv16, pack_bias(b3),
                   batch_pk, Wl, bl.reshape(1, C))
